# Initial kernel scaffold; baseline (speedup 1.0000x reference)
import functools

import jax
import jax.numpy as jnp
from jax import lax
from jax.experimental import pallas as pl
from jax.experimental.pallas import tpu as pltpu
from jax.experimental.pallas import tpu_sc as plsc

_N = 2048
_B = 8
_E = 12288
_NTILES = 32
_CHUNK_ROWS = 32
_LANES = 16

_DIMS = [1536, 512, 512, 256, 256, 128, 128]


def _adj_body(edges_hbm, out_hbm, src_v, dst_v, block_v):
    wid = lax.axis_index("s") * 2 + lax.axis_index("c")
    pltpu.sync_copy(edges_hbm.at[0], src_v)
    pltpu.sync_copy(edges_hbm.at[1], dst_v)

    big = jnp.int32(2147483647)
    pos = lax.iota(jnp.int32, _LANES)
    ones = jnp.ones((_LANES,), jnp.float32)

    for c in range(2):
        base = (wid * 2 + c) * _CHUNK_ROWS

        def zero_body(i, carry):
            block_v[pl.ds(i * _LANES, _LANES)] = jnp.zeros((_LANES,),
                                                           jnp.float32)
            return carry

        lax.fori_loop(0, _CHUNK_ROWS * _N // _LANES, zero_body, 0, unroll=8)

        def edge_body(i, carry):
            d = dst_v[pl.ds(i * _LANES, _LANES)]
            s = src_v[pl.ds(i * _LANES, _LANES)]
            valid = (d >= base) & (d < base + _CHUNK_ROWS)

            @pl.when(jnp.any(valid))
            def _():
                local = (d - base) * _N + s
                key = jnp.where(valid, local, big)
                skey = jnp.sort(key)
                prev = skey.at[jnp.maximum(pos - 1, 0)].get(
                    mode="promise_in_bounds")
                nxt = skey.at[jnp.minimum(pos + 1, _LANES - 1)].get(
                    mode="promise_in_bounds")
                is_start = (skey != prev) | (pos == 0)
                is_end = ((skey != nxt) | (pos == _LANES - 1)) & (skey != big)
                run_start = plsc.cummax(jnp.where(is_start, pos, 0))
                cnt = (pos - run_start + 1).astype(jnp.float32)
                idxs = jnp.where(is_end, skey, 0)
                plsc.addupdate_scatter(block_v, [idxs], cnt, mask=is_end)

            return carry

        lax.fori_loop(0, _E // _LANES, edge_body, 0)

        for j in range(2):
            r = pos + j * _LANES
            plsc.addupdate_scatter(block_v, [r * _N + (base + r)], ones)

        pltpu.sync_copy(block_v,
                        out_hbm.at[pl.ds(base * _N, _CHUNK_ROWS * _N)])


def _build_adj(edges):
    k = pl.kernel(
        _adj_body,
        out_type=jax.ShapeDtypeStruct((_N * _N,), jnp.float32),
        mesh=plsc.VectorSubcoreMesh(core_axis_name="c", subcore_axis_name="s"),
        scratch_types=[
            pltpu.VMEM((_E,), jnp.int32),
            pltpu.VMEM((_E,), jnp.int32),
            pltpu.VMEM((_CHUNK_ROWS * _N,), jnp.float32),
        ],
    )
    return k(edges).reshape(_N, _N)


def _deg_kernel(adj_ref, o_ref):
    deg = jnp.sum(adj_ref[...], axis=1)
    o_ref[0, 0, :] = lax.rsqrt(deg)


def _compute_dinv(adj):
    out = pl.pallas_call(
        _deg_kernel,
        grid=(_N // 128,),
        in_specs=[pl.BlockSpec((128, _N), lambda i: (i, 0))],
        out_specs=pl.BlockSpec((1, 1, 128), lambda i: (i, 0, 0)),
        out_shape=jax.ShapeDtypeStruct((_N // 128, 1, 128), jnp.float32),
    )(adj)
    return out.reshape(_N)


def _scale_kernel(adj_ref, dcol_ref, drow_ref, o_ref):
    o_ref[...] = adj_ref[...] * dcol_ref[:, 0:1] * drow_ref[...]


def _normalize_adj(adj, dinv):
    dcol = jnp.broadcast_to(dinv.reshape(_N, 1), (_N, 128))
    drow = dinv.reshape(1, _N)
    return pl.pallas_call(
        _scale_kernel,
        grid=(_N // 128,),
        in_specs=[
            pl.BlockSpec((128, _N), lambda i: (i, 0)),
            pl.BlockSpec((128, 128), lambda i: (i, 0)),
            pl.BlockSpec((1, _N), lambda i: (0, 0)),
        ],
        out_specs=pl.BlockSpec((128, _N), lambda i: (i, 0)),
        out_shape=jax.ShapeDtypeStruct((_N, _N), jnp.float32),
    )(adj, dcol, drow)


def _xw_kernel(x_ref, w_ref, o_ref):
    o_ref[...] = jnp.dot(x_ref[...], w_ref[...],
                         preferred_element_type=jnp.float32)


def _dense(x, w, bm=1024):
    m, kdim = x.shape
    _, n = w.shape
    return pl.pallas_call(
        _xw_kernel,
        grid=(m // bm,),
        in_specs=[
            pl.BlockSpec((bm, kdim), lambda i: (i, 0)),
            pl.BlockSpec((kdim, n), lambda i: (0, 0)),
        ],
        out_specs=pl.BlockSpec((bm, n), lambda i: (i, 0)),
        out_shape=jax.ShapeDtypeStruct((m, n), jnp.float32),
        compiler_params=pltpu.CompilerParams(
            dimension_semantics=("arbitrary",)),
    )(x, w)


def _leaky(v):
    return jnp.where(v >= 0, v, 0.01 * v)


def _agg_kernel(a_ref, h_ref, b_ref, o_ref, *, act):
    v = jnp.dot(a_ref[...], h_ref[0], preferred_element_type=jnp.float32)
    v = v + b_ref[...]
    o_ref[0] = _leaky(v) if act else v


def _agg(a, h, b, act):
    n = h.shape[-1]
    return pl.pallas_call(
        functools.partial(_agg_kernel, act=act),
        grid=(_B,),
        in_specs=[
            pl.BlockSpec((_N, _N), lambda i: (0, 0)),
            pl.BlockSpec((1, _N, n), lambda i: (i, 0, 0)),
            pl.BlockSpec((1, n), lambda i: (0, 0)),
        ],
        out_specs=pl.BlockSpec((1, _N, n), lambda i: (i, 0, 0)),
        out_shape=jax.ShapeDtypeStruct((_B, _N, n), jnp.float32),
        compiler_params=pltpu.CompilerParams(
            dimension_semantics=("arbitrary",)),
    )(a, h, b)


def _dec_kernel(f_ref, w_ref, b_ref, o_ref):
    v = jnp.dot(f_ref[...], w_ref[...], preferred_element_type=jnp.float32)
    o_ref[...] = jnp.tanh(v + b_ref[...]) * 0.1


def _decoder(feats, w_dec, b_dec, bn=512):
    m, kdim = feats.shape
    return pl.pallas_call(
        _dec_kernel,
        grid=(kdim // bn,),
        in_specs=[
            pl.BlockSpec((m, kdim), lambda i: (0, 0)),
            pl.BlockSpec((kdim, bn), lambda i: (0, i)),
            pl.BlockSpec((1, bn), lambda i: (0, i)),
        ],
        out_specs=pl.BlockSpec((m, bn), lambda i: (0, i)),
        out_shape=jax.ShapeDtypeStruct((m, kdim), jnp.float32),
        compiler_params=pltpu.CompilerParams(
            dimension_semantics=("arbitrary",)),
    )(feats, w_dec, b_dec)


def _pad2(w, rows, cols):
    r, c = w.shape
    return jnp.pad(w, ((0, rows - r), (0, cols - c)))


def kernel(batch_vertices, local_features, global_features, edges,
           W0, b0, W1, b1, W2, b2, W3, b3, W4, b4, W5, b5, W_dec, b_dec):
    adj = _build_adj(edges)
    dinv = _compute_dinv(adj)
    a = _normalize_adj(adj, dinv)

    gf = jnp.broadcast_to(global_features[:, None, :],
                          (_B, _N, global_features.shape[-1]))
    x = jnp.concatenate([batch_vertices, local_features, gf], axis=2)
    x = jnp.pad(x, ((0, 0), (0, 0), (0, _DIMS[0] - x.shape[-1])))

    ws = [W0, W1, W2, W3, W4, W5]
    bs = [b0, b1, b2, b3, b4, b5]
    for i in range(6):
        kin, kout = _DIMS[i], _DIMS[i + 1]
        w = _pad2(ws[i], kin, kout)
        b = jnp.pad(bs[i], (0, kout - bs[i].shape[0])).reshape(1, kout)
        h = _dense(x.reshape(_B * _N, kin), w)
        x = _agg(a, h.reshape(_B, _N, kout), b, act=(i % 2 == 1))

    feats = x[..., :3].reshape(_B, _N * 3)
    out = _decoder(feats, W_dec, b_dec.reshape(1, _N * 3))
    return out.reshape(_B, _N, 3)


# speedup vs baseline: 14.1153x; 14.1153x over previous
import functools

import jax
import jax.numpy as jnp
from jax import lax
from jax.experimental import pallas as pl
from jax.experimental.pallas import tpu as pltpu
from jax.experimental.pallas import tpu_sc as plsc

_N = 2048
_B = 8
_E = 12288
_NTILES = 32
_CHUNK_ROWS = 32
_LANES = 16

_DIMS = [1536, 512, 512, 256, 256, 128, 128]


def _adj_body(edges_hbm, out_hbm, src_v, dst_v, block_v):
    wid = lax.axis_index("s") * 2 + lax.axis_index("c")
    pltpu.sync_copy(edges_hbm.at[0], src_v)
    pltpu.sync_copy(edges_hbm.at[1], dst_v)

    big = jnp.int32(2147483647)
    pos = lax.iota(jnp.int32, _LANES)
    ones = jnp.ones((_LANES,), jnp.float32)

    for c in range(2):
        base = (wid * 2 + c) * _CHUNK_ROWS

        def zero_body(i, carry):
            block_v[pl.ds(i * _LANES, _LANES)] = jnp.zeros((_LANES,),
                                                           jnp.float32)
            return carry

        lax.fori_loop(0, _CHUNK_ROWS * _N // _LANES, zero_body, 0, unroll=8)

        def edge_body(i, carry):
            d = dst_v[pl.ds(i * _LANES, _LANES)]
            s = src_v[pl.ds(i * _LANES, _LANES)]
            valid = (d >= base) & (d < base + _CHUNK_ROWS)

            @pl.when(jnp.any(valid))
            def _():
                local = (d - base) * _N + s
                key = jnp.where(valid, local, big)
                skey = jnp.sort(key)
                prev = skey.at[jnp.maximum(pos - 1, 0)].get(
                    mode="promise_in_bounds")
                nxt = skey.at[jnp.minimum(pos + 1, _LANES - 1)].get(
                    mode="promise_in_bounds")
                is_start = (skey != prev) | (pos == 0)
                is_end = ((skey != nxt) | (pos == _LANES - 1)) & (skey != big)
                run_start = jnp.where(is_start, pos, 0)
                for sh in (1, 2, 4, 8):
                    shifted = run_start.at[jnp.maximum(pos - sh, 0)].get(
                        mode="promise_in_bounds")
                    run_start = jnp.maximum(
                        run_start, jnp.where(pos >= sh, shifted, 0))
                cnt = (pos - run_start + 1).astype(jnp.float32)
                idxs = jnp.where(is_end, skey, 0)
                plsc.addupdate_scatter(block_v, [idxs], cnt, mask=is_end)

            return carry

        lax.fori_loop(0, _E // _LANES, edge_body, 0)

        for j in range(2):
            r = pos + j * _LANES
            plsc.addupdate_scatter(block_v, [r * _N + (base + r)], ones)

        pltpu.sync_copy(block_v,
                        out_hbm.at[pl.ds(base * _N, _CHUNK_ROWS * _N)])


def _build_adj(edges):
    k = pl.kernel(
        _adj_body,
        out_type=jax.ShapeDtypeStruct((_N * _N,), jnp.float32),
        mesh=plsc.VectorSubcoreMesh(core_axis_name="c", subcore_axis_name="s"),
        compiler_params=pltpu.CompilerParams(needs_layout_passes=False),
        scratch_types=[
            pltpu.VMEM((_E,), jnp.int32),
            pltpu.VMEM((_E,), jnp.int32),
            pltpu.VMEM((_CHUNK_ROWS * _N,), jnp.float32),
        ],
    )
    return k(edges).reshape(_N, _N)


def _deg_kernel(adj_ref, o_ref):
    deg = jnp.sum(adj_ref[...], axis=1)
    o_ref[0, 0, :] = lax.rsqrt(deg)


def _compute_dinv(adj):
    out = pl.pallas_call(
        _deg_kernel,
        grid=(_N // 128,),
        in_specs=[pl.BlockSpec((128, _N), lambda i: (i, 0))],
        out_specs=pl.BlockSpec((1, 1, 128), lambda i: (i, 0, 0)),
        out_shape=jax.ShapeDtypeStruct((_N // 128, 1, 128), jnp.float32),
    )(adj)
    return out.reshape(_N)


def _scale_kernel(adj_ref, dcol_ref, drow_ref, o_ref):
    o_ref[...] = adj_ref[...] * dcol_ref[:, 0:1] * drow_ref[...]


def _normalize_adj(adj, dinv):
    dcol = jnp.broadcast_to(dinv.reshape(_N, 1), (_N, 128))
    drow = dinv.reshape(1, _N)
    return pl.pallas_call(
        _scale_kernel,
        grid=(_N // 128,),
        in_specs=[
            pl.BlockSpec((128, _N), lambda i: (i, 0)),
            pl.BlockSpec((128, 128), lambda i: (i, 0)),
            pl.BlockSpec((1, _N), lambda i: (0, 0)),
        ],
        out_specs=pl.BlockSpec((128, _N), lambda i: (i, 0)),
        out_shape=jax.ShapeDtypeStruct((_N, _N), jnp.float32),
    )(adj, dcol, drow)


def _xw_kernel(x_ref, w_ref, o_ref):
    o_ref[...] = jnp.dot(x_ref[...], w_ref[...],
                         preferred_element_type=jnp.float32)


def _dense(x, w, bm=1024):
    m, kdim = x.shape
    _, n = w.shape
    return pl.pallas_call(
        _xw_kernel,
        grid=(m // bm,),
        in_specs=[
            pl.BlockSpec((bm, kdim), lambda i: (i, 0)),
            pl.BlockSpec((kdim, n), lambda i: (0, 0)),
        ],
        out_specs=pl.BlockSpec((bm, n), lambda i: (i, 0)),
        out_shape=jax.ShapeDtypeStruct((m, n), jnp.float32),
        compiler_params=pltpu.CompilerParams(
            dimension_semantics=("arbitrary",)),
    )(x, w)


def _leaky(v):
    return jnp.where(v >= 0, v, 0.01 * v)


def _agg_kernel(a_ref, h_ref, b_ref, o_ref, *, act):
    v = jnp.dot(a_ref[...], h_ref[0], preferred_element_type=jnp.float32)
    v = v + b_ref[...]
    o_ref[0] = _leaky(v) if act else v


def _agg(a, h, b, act):
    n = h.shape[-1]
    return pl.pallas_call(
        functools.partial(_agg_kernel, act=act),
        grid=(_B,),
        in_specs=[
            pl.BlockSpec((_N, _N), lambda i: (0, 0)),
            pl.BlockSpec((1, _N, n), lambda i: (i, 0, 0)),
            pl.BlockSpec((1, n), lambda i: (0, 0)),
        ],
        out_specs=pl.BlockSpec((1, _N, n), lambda i: (i, 0, 0)),
        out_shape=jax.ShapeDtypeStruct((_B, _N, n), jnp.float32),
        compiler_params=pltpu.CompilerParams(
            dimension_semantics=("arbitrary",)),
    )(a, h, b)


def _dec_kernel(f_ref, w_ref, b_ref, o_ref):
    v = jnp.dot(f_ref[...], w_ref[...], preferred_element_type=jnp.float32)
    o_ref[...] = jnp.tanh(v + b_ref[...]) * 0.1


def _decoder(feats, w_dec, b_dec, bn=512):
    m, kdim = feats.shape
    return pl.pallas_call(
        _dec_kernel,
        grid=(kdim // bn,),
        in_specs=[
            pl.BlockSpec((m, kdim), lambda i: (0, 0)),
            pl.BlockSpec((kdim, bn), lambda i: (0, i)),
            pl.BlockSpec((1, bn), lambda i: (0, i)),
        ],
        out_specs=pl.BlockSpec((m, bn), lambda i: (0, i)),
        out_shape=jax.ShapeDtypeStruct((m, kdim), jnp.float32),
        compiler_params=pltpu.CompilerParams(
            dimension_semantics=("arbitrary",)),
    )(feats, w_dec, b_dec)


def _pad2(w, rows, cols):
    r, c = w.shape
    return jnp.pad(w, ((0, rows - r), (0, cols - c)))


def kernel(batch_vertices, local_features, global_features, edges,
           W0, b0, W1, b1, W2, b2, W3, b3, W4, b4, W5, b5, W_dec, b_dec):
    adj = _build_adj(edges)
    dinv = _compute_dinv(adj)
    a = _normalize_adj(adj, dinv)

    gf = jnp.broadcast_to(global_features[:, None, :],
                          (_B, _N, global_features.shape[-1]))
    x = jnp.concatenate([batch_vertices, local_features, gf], axis=2)
    x = jnp.pad(x, ((0, 0), (0, 0), (0, _DIMS[0] - x.shape[-1])))

    ws = [W0, W1, W2, W3, W4, W5]
    bs = [b0, b1, b2, b3, b4, b5]
    for i in range(6):
        kin, kout = _DIMS[i], _DIMS[i + 1]
        w = _pad2(ws[i], kin, kout)
        b = jnp.pad(bs[i], (0, kout - bs[i].shape[0])).reshape(1, kout)
        h = _dense(x.reshape(_B * _N, kin), w)
        x = _agg(a, h.reshape(_B, _N, kout), b, act=(i % 2 == 1))

    feats = x[..., :3].reshape(_B, _N * 3)
    out = _decoder(feats, W_dec, b_dec.reshape(1, _N * 3))
    return out.reshape(_B, _N, 3)


# speedup vs baseline: 15.5328x; 1.1004x over previous
import functools

import jax
import jax.numpy as jnp
from jax import lax
from jax.experimental import pallas as pl
from jax.experimental.pallas import tpu as pltpu
from jax.experimental.pallas import tpu_sc as plsc

_N = 2048
_B = 8
_E = 12288
_NTILES = 32
_CHUNK_ROWS = 32
_LANES = 16

_DIMS = [1536, 512, 512, 256, 256, 128, 128]


def _adj_body(edges_hbm, out_hbm, src_v, dst_v, block_v):
    wid = lax.axis_index("s") * 2 + lax.axis_index("c")
    pltpu.sync_copy(edges_hbm.at[0], src_v)
    pltpu.sync_copy(edges_hbm.at[1], dst_v)

    big = jnp.int32(2147483647)
    pos = lax.iota(jnp.int32, _LANES)
    ones = jnp.ones((_LANES,), jnp.float32)

    for c in range(2):
        base = (wid * 2 + c) * _CHUNK_ROWS

        def zero_body(i, carry):
            block_v[pl.ds(i * _LANES, _LANES)] = jnp.zeros((_LANES,),
                                                           jnp.float32)
            return carry

        lax.fori_loop(0, _CHUNK_ROWS * _N // _LANES, zero_body, 0, unroll=8)

        def edge_body(i, carry):
            d = dst_v[pl.ds(i * _LANES, _LANES)]
            s = src_v[pl.ds(i * _LANES, _LANES)]
            valid = (d >= base) & (d < base + _CHUNK_ROWS)

            @pl.when(jnp.any(valid))
            def _():
                local = (d - base) * _N + s
                key = jnp.where(valid, local, big)
                skey = jnp.sort(key)
                prev = skey.at[jnp.maximum(pos - 1, 0)].get(
                    mode="promise_in_bounds")
                nxt = skey.at[jnp.minimum(pos + 1, _LANES - 1)].get(
                    mode="promise_in_bounds")
                is_start = (skey != prev) | (pos == 0)
                is_end = ((skey != nxt) | (pos == _LANES - 1)) & (skey != big)
                run_start = jnp.where(is_start, pos, 0)
                for sh in (1, 2, 4, 8):
                    shifted = run_start.at[jnp.maximum(pos - sh, 0)].get(
                        mode="promise_in_bounds")
                    run_start = jnp.maximum(
                        run_start, jnp.where(pos >= sh, shifted, 0))
                cnt = (pos - run_start + 1).astype(jnp.float32)
                idxs = jnp.where(is_end, skey, 0)
                plsc.addupdate_scatter(block_v, [idxs], cnt, mask=is_end)

            return carry

        lax.fori_loop(0, _E // _LANES, edge_body, 0)

        for j in range(2):
            r = pos + j * _LANES
            plsc.addupdate_scatter(block_v, [r * _N + (base + r)], ones)

        pltpu.sync_copy(block_v,
                        out_hbm.at[pl.ds(base * _N, _CHUNK_ROWS * _N)])


def _build_adj(edges):
    k = pl.kernel(
        _adj_body,
        out_type=jax.ShapeDtypeStruct((_N * _N,), jnp.float32),
        mesh=plsc.VectorSubcoreMesh(core_axis_name="c", subcore_axis_name="s"),
        compiler_params=pltpu.CompilerParams(needs_layout_passes=False),
        scratch_types=[
            pltpu.VMEM((_E,), jnp.int32),
            pltpu.VMEM((_E,), jnp.int32),
            pltpu.VMEM((_CHUNK_ROWS * _N,), jnp.float32),
        ],
    )
    return k(edges).reshape(_N, _N)


def _deg_kernel(adj_ref, o_ref):
    deg = jnp.sum(adj_ref[...], axis=1)
    o_ref[0, 0, :] = lax.rsqrt(deg)


def _compute_dinv(adj):
    out = pl.pallas_call(
        _deg_kernel,
        grid=(_N // 128,),
        in_specs=[pl.BlockSpec((128, _N), lambda i: (i, 0))],
        out_specs=pl.BlockSpec((1, 1, 128), lambda i: (i, 0, 0)),
        out_shape=jax.ShapeDtypeStruct((_N // 128, 1, 128), jnp.float32),
    )(adj)
    return out.reshape(_N)


def _scale_kernel(adj_ref, dcol_ref, drow_ref, o_ref):
    v = adj_ref[...] * dcol_ref[:, 0:1] * drow_ref[...]
    o_ref[...] = v.astype(jnp.bfloat16)


def _normalize_adj(adj, dinv):
    dcol = jnp.broadcast_to(dinv.reshape(_N, 1), (_N, 128))
    drow = dinv.reshape(1, _N)
    return pl.pallas_call(
        _scale_kernel,
        grid=(_N // 128,),
        in_specs=[
            pl.BlockSpec((128, _N), lambda i: (i, 0)),
            pl.BlockSpec((128, 128), lambda i: (i, 0)),
            pl.BlockSpec((1, _N), lambda i: (0, 0)),
        ],
        out_specs=pl.BlockSpec((128, _N), lambda i: (i, 0)),
        out_shape=jax.ShapeDtypeStruct((_N, _N), jnp.bfloat16),
    )(adj, dcol, drow)


def _xw_kernel(x_ref, w_ref, o_ref):
    v = jnp.dot(x_ref[...], w_ref[...], preferred_element_type=jnp.float32)
    o_ref[...] = v.astype(jnp.bfloat16)


def _dense(x, w, bm=1024):
    m, kdim = x.shape
    _, n = w.shape
    return pl.pallas_call(
        _xw_kernel,
        grid=(m // bm,),
        in_specs=[
            pl.BlockSpec((bm, kdim), lambda i: (i, 0)),
            pl.BlockSpec((kdim, n), lambda i: (0, 0)),
        ],
        out_specs=pl.BlockSpec((bm, n), lambda i: (i, 0)),
        out_shape=jax.ShapeDtypeStruct((m, n), jnp.bfloat16),
        compiler_params=pltpu.CompilerParams(
            dimension_semantics=("arbitrary",)),
    )(x, w)


def _leaky(v):
    return jnp.where(v >= 0, v, 0.01 * v)


def _agg_kernel(a_ref, h_ref, b_ref, o_ref, *, act):
    v = jnp.dot(a_ref[...], h_ref[0], preferred_element_type=jnp.float32)
    v = v + b_ref[...]
    v = _leaky(v) if act else v
    o_ref[0] = v.astype(o_ref.dtype)


def _agg(a, h, b, act, out_dtype=jnp.bfloat16):
    n = h.shape[-1]
    return pl.pallas_call(
        functools.partial(_agg_kernel, act=act),
        grid=(_B,),
        in_specs=[
            pl.BlockSpec((_N, _N), lambda i: (0, 0)),
            pl.BlockSpec((1, _N, n), lambda i: (i, 0, 0)),
            pl.BlockSpec((1, n), lambda i: (0, 0)),
        ],
        out_specs=pl.BlockSpec((1, _N, n), lambda i: (i, 0, 0)),
        out_shape=jax.ShapeDtypeStruct((_B, _N, n), out_dtype),
        compiler_params=pltpu.CompilerParams(
            dimension_semantics=("arbitrary",)),
    )(a, h, b)


def _dec_kernel(f_ref, w_ref, b_ref, o_ref):
    v = jnp.dot(f_ref[...], w_ref[...], preferred_element_type=jnp.float32)
    o_ref[...] = jnp.tanh(v + b_ref[...]) * 0.1


def _decoder(feats, w_dec, b_dec, bn=512):
    m, kdim = feats.shape
    return pl.pallas_call(
        _dec_kernel,
        grid=(kdim // bn,),
        in_specs=[
            pl.BlockSpec((m, kdim), lambda i: (0, 0)),
            pl.BlockSpec((kdim, bn), lambda i: (0, i)),
            pl.BlockSpec((1, bn), lambda i: (0, i)),
        ],
        out_specs=pl.BlockSpec((m, bn), lambda i: (0, i)),
        out_shape=jax.ShapeDtypeStruct((m, kdim), jnp.float32),
        compiler_params=pltpu.CompilerParams(
            dimension_semantics=("arbitrary",)),
    )(feats, w_dec, b_dec)


def _pad2(w, rows, cols):
    r, c = w.shape
    return jnp.pad(w, ((0, rows - r), (0, cols - c)))


def kernel(batch_vertices, local_features, global_features, edges,
           W0, b0, W1, b1, W2, b2, W3, b3, W4, b4, W5, b5, W_dec, b_dec):
    adj = _build_adj(edges)
    dinv = _compute_dinv(adj)
    a = _normalize_adj(adj, dinv)

    gf = jnp.broadcast_to(global_features[:, None, :].astype(jnp.bfloat16),
                          (_B, _N, global_features.shape[-1]))
    x = jnp.concatenate([batch_vertices.astype(jnp.bfloat16),
                         local_features.astype(jnp.bfloat16), gf], axis=2)
    x = jnp.pad(x, ((0, 0), (0, 0), (0, _DIMS[0] - x.shape[-1])))

    ws = [W0, W1, W2, W3, W4, W5]
    bs = [b0, b1, b2, b3, b4, b5]
    for i in range(6):
        kin, kout = _DIMS[i], _DIMS[i + 1]
        w = _pad2(ws[i], kin, kout).astype(jnp.bfloat16)
        b = jnp.pad(bs[i], (0, kout - bs[i].shape[0])).reshape(1, kout)
        h = _dense(x.reshape(_B * _N, kin), w)
        x = _agg(a, h.reshape(_B, _N, kout), b, act=(i % 2 == 1),
                 out_dtype=(jnp.float32 if i == 5 else jnp.bfloat16))

    feats = x[..., :3].reshape(_B, _N * 3)
    out = _decoder(feats, W_dec, b_dec.reshape(1, _N * 3))
    return out.reshape(_B, _N, 3)


# speedup vs baseline: 23.8590x; 1.5360x over previous
import functools

import jax
import jax.numpy as jnp
from jax import lax
from jax.experimental import pallas as pl
from jax.experimental.pallas import tpu as pltpu
from jax.experimental.pallas import tpu_sc as plsc

_N = 2048
_B = 8
_E = 12288
_NTILES = 32
_CHUNK_ROWS = 32
_LANES = 16

_DIMS = [1536, 512, 512, 256, 256, 128, 128]


def _adj_body(edges_hbm, out_hbm, src_v, dst_v, block_v):
    wid = lax.axis_index("s") * 2 + lax.axis_index("c")
    pltpu.sync_copy(edges_hbm.at[0], src_v)
    pltpu.sync_copy(edges_hbm.at[1], dst_v)

    big = jnp.int32(2147483647)
    pos = lax.iota(jnp.int32, _LANES)
    ones = jnp.ones((_LANES,), jnp.float32)

    for c in range(2):
        base = (wid * 2 + c) * _CHUNK_ROWS

        def zero_body(i, carry):
            block_v[i >> 7, pl.ds((i & 127) * _LANES, _LANES)] = (
                jnp.zeros((_LANES,), jnp.float32))
            return carry

        lax.fori_loop(0, _CHUNK_ROWS * _N // _LANES, zero_body, 0, unroll=8)

        def edge_body(i, carry):
            d = dst_v[pl.ds(i * _LANES, _LANES)]
            s = src_v[pl.ds(i * _LANES, _LANES)]
            valid = (d >= base) & (d < base + _CHUNK_ROWS)

            @pl.when(jnp.any(valid))
            def _():
                local = (d - base) * _N + s
                key = jnp.where(valid, local, big)
                skey = jnp.sort(key)
                prev = skey.at[jnp.maximum(pos - 1, 0)].get(
                    mode="promise_in_bounds")
                nxt = skey.at[jnp.minimum(pos + 1, _LANES - 1)].get(
                    mode="promise_in_bounds")
                is_start = (skey != prev) | (pos == 0)
                is_end = ((skey != nxt) | (pos == _LANES - 1)) & (skey != big)
                run_start = jnp.where(is_start, pos, 0)
                for sh in (1, 2, 4, 8):
                    shifted = run_start.at[jnp.maximum(pos - sh, 0)].get(
                        mode="promise_in_bounds")
                    run_start = jnp.maximum(
                        run_start, jnp.where(pos >= sh, shifted, 0))
                cnt = (pos - run_start + 1).astype(jnp.float32)
                rows = jnp.where(is_end, lax.shift_right_logical(skey, 11), 0)
                cols = jnp.where(is_end, jnp.bitwise_and(skey, _N - 1), 0)
                plsc.addupdate_scatter(block_v, [rows, cols], cnt,
                                       mask=is_end)

            return carry

        lax.fori_loop(0, _E // _LANES, edge_body, 0)

        for j in range(2):
            r = pos + j * _LANES
            plsc.addupdate_scatter(block_v, [r, base + r], ones)

        pltpu.sync_copy(block_v, out_hbm.at[pl.ds(base, _CHUNK_ROWS)])


def _build_adj(edges):
    k = pl.kernel(
        _adj_body,
        out_type=jax.ShapeDtypeStruct((_N, _N), jnp.float32),
        mesh=plsc.VectorSubcoreMesh(core_axis_name="c", subcore_axis_name="s"),
        compiler_params=pltpu.CompilerParams(needs_layout_passes=False),
        scratch_types=[
            pltpu.VMEM((_E,), jnp.int32),
            pltpu.VMEM((_E,), jnp.int32),
            pltpu.VMEM((_CHUNK_ROWS, _N), jnp.float32),
        ],
    )
    return k(edges)


def _deg_kernel(adj_ref, o_ref):
    deg = jnp.sum(adj_ref[...], axis=1)
    o_ref[0, 0, :] = lax.rsqrt(deg)


def _compute_dinv(adj):
    out = pl.pallas_call(
        _deg_kernel,
        grid=(_N // 128,),
        in_specs=[pl.BlockSpec((128, _N), lambda i: (i, 0))],
        out_specs=pl.BlockSpec((1, 1, 128), lambda i: (i, 0, 0)),
        out_shape=jax.ShapeDtypeStruct((_N // 128, 1, 128), jnp.float32),
    )(adj)
    return out.reshape(_N)


def _scale_kernel(adj_ref, dcol_ref, drow_ref, o_ref):
    v = adj_ref[...] * dcol_ref[:, 0:1] * drow_ref[...]
    o_ref[...] = v.astype(jnp.bfloat16)


def _normalize_adj(adj, dinv):
    dcol = jnp.broadcast_to(dinv.reshape(_N, 1), (_N, 128))
    drow = dinv.reshape(1, _N)
    return pl.pallas_call(
        _scale_kernel,
        grid=(_N // 128,),
        in_specs=[
            pl.BlockSpec((128, _N), lambda i: (i, 0)),
            pl.BlockSpec((128, 128), lambda i: (i, 0)),
            pl.BlockSpec((1, _N), lambda i: (0, 0)),
        ],
        out_specs=pl.BlockSpec((128, _N), lambda i: (i, 0)),
        out_shape=jax.ShapeDtypeStruct((_N, _N), jnp.bfloat16),
    )(adj, dcol, drow)


def _xw_kernel(x_ref, w_ref, o_ref):
    v = jnp.dot(x_ref[...], w_ref[...], preferred_element_type=jnp.float32)
    o_ref[...] = v.astype(jnp.bfloat16)


def _dense(x, w, bm=1024):
    m, kdim = x.shape
    _, n = w.shape
    return pl.pallas_call(
        _xw_kernel,
        grid=(m // bm,),
        in_specs=[
            pl.BlockSpec((bm, kdim), lambda i: (i, 0)),
            pl.BlockSpec((kdim, n), lambda i: (0, 0)),
        ],
        out_specs=pl.BlockSpec((bm, n), lambda i: (i, 0)),
        out_shape=jax.ShapeDtypeStruct((m, n), jnp.bfloat16),
        compiler_params=pltpu.CompilerParams(
            dimension_semantics=("arbitrary",)),
    )(x, w)


def _gm_kernel(g_ref, w_ref, o_ref):
    v = jnp.dot(g_ref[...].astype(jnp.bfloat16), w_ref[...],
                preferred_element_type=jnp.float32)
    o_ref[...] = v


def _global_matmul(gf, w0c):
    return pl.pallas_call(
        _gm_kernel,
        out_shape=jax.ShapeDtypeStruct((_B, w0c.shape[1]), jnp.float32),
    )(gf, w0c)


def _l0_kernel(lf_ref, bv_ref, gm_ref, wb_ref, wa_ref, o_ref):
    x = lf_ref[...].astype(jnp.bfloat16)
    v = jnp.dot(x, wb_ref[...], preferred_element_type=jnp.float32)
    bv = bv_ref[...]
    for c in range(3):
        v += bv[:, c:c + 1] * wa_ref[c:c + 1, :]
    v += gm_ref[0]
    o_ref[...] = v.astype(jnp.bfloat16)


def _layer0(lf, bv, gm, w0b, w0a, bm=1024):
    m, kdim = lf.shape
    n = w0b.shape[1]
    return pl.pallas_call(
        _l0_kernel,
        grid=(m // bm,),
        in_specs=[
            pl.BlockSpec((bm, kdim), lambda i: (i, 0)),
            pl.BlockSpec((bm, 3), lambda i: (i, 0)),
            pl.BlockSpec((1, 1, n), lambda i: (i * bm // _N, 0, 0)),
            pl.BlockSpec((kdim, n), lambda i: (0, 0)),
            pl.BlockSpec((3, n), lambda i: (0, 0)),
        ],
        out_specs=pl.BlockSpec((bm, n), lambda i: (i, 0)),
        out_shape=jax.ShapeDtypeStruct((m, n), jnp.bfloat16),
        compiler_params=pltpu.CompilerParams(
            dimension_semantics=("arbitrary",)),
    )(lf, bv, gm, w0b, w0a)


def _leaky(v):
    return jnp.where(v >= 0, v, 0.01 * v)


def _agg_kernel(a_ref, h_ref, b_ref, o_ref, *, act):
    v = jnp.dot(a_ref[...], h_ref[0], preferred_element_type=jnp.float32)
    v = v + b_ref[...]
    v = _leaky(v) if act else v
    o_ref[0] = v.astype(o_ref.dtype)


def _agg(a, h, b, act, out_dtype=jnp.bfloat16):
    n = h.shape[-1]
    return pl.pallas_call(
        functools.partial(_agg_kernel, act=act),
        grid=(_B,),
        in_specs=[
            pl.BlockSpec((_N, _N), lambda i: (0, 0)),
            pl.BlockSpec((1, _N, n), lambda i: (i, 0, 0)),
            pl.BlockSpec((1, n), lambda i: (0, 0)),
        ],
        out_specs=pl.BlockSpec((1, _N, n), lambda i: (i, 0, 0)),
        out_shape=jax.ShapeDtypeStruct((_B, _N, n), out_dtype),
        compiler_params=pltpu.CompilerParams(
            dimension_semantics=("arbitrary",)),
    )(a, h, b)


def _dec_kernel(f_ref, w_ref, b_ref, o_ref):
    v = jnp.dot(f_ref[...], w_ref[...], preferred_element_type=jnp.float32)
    o_ref[...] = jnp.tanh(v + b_ref[...]) * 0.1


def _decoder(feats, w_dec, b_dec, bn=512):
    m, kdim = feats.shape
    return pl.pallas_call(
        _dec_kernel,
        grid=(kdim // bn,),
        in_specs=[
            pl.BlockSpec((m, kdim), lambda i: (0, 0)),
            pl.BlockSpec((kdim, bn), lambda i: (0, i)),
            pl.BlockSpec((1, bn), lambda i: (0, i)),
        ],
        out_specs=pl.BlockSpec((m, bn), lambda i: (0, i)),
        out_shape=jax.ShapeDtypeStruct((m, kdim), jnp.float32),
        compiler_params=pltpu.CompilerParams(
            dimension_semantics=("arbitrary",)),
    )(feats, w_dec, b_dec)


def _pad2(w, rows, cols):
    r, c = w.shape
    return jnp.pad(w, ((0, rows - r), (0, cols - c)))


def kernel(batch_vertices, local_features, global_features, edges,
           W0, b0, W1, b1, W2, b2, W3, b3, W4, b4, W5, b5, W_dec, b_dec):
    adj = _build_adj(edges)
    dinv = _compute_dinv(adj)
    a = _normalize_adj(adj, dinv)

    nf = local_features.shape[-1]
    w0a = W0[:3]
    w0b = W0[3:3 + nf].astype(jnp.bfloat16)
    w0c = W0[3 + nf:].astype(jnp.bfloat16)
    gm = _global_matmul(global_features, w0c).reshape(_B, 1, _DIMS[1])
    h = _layer0(local_features.reshape(_B * _N, nf),
                batch_vertices.reshape(_B * _N, 3), gm, w0b, w0a)
    x = _agg(a, h.reshape(_B, _N, _DIMS[1]), b0.reshape(1, -1), act=False)

    ws = [W1, W2, W3, W4, W5]
    bs = [b1, b2, b3, b4, b5]
    for i in range(1, 6):
        kin, kout = _DIMS[i], _DIMS[i + 1]
        w = _pad2(ws[i - 1], kin, kout).astype(jnp.bfloat16)
        b = jnp.pad(bs[i - 1],
                    (0, kout - bs[i - 1].shape[0])).reshape(1, kout)
        h = _dense(x.reshape(_B * _N, kin), w)
        x = _agg(a, h.reshape(_B, _N, kout), b, act=(i % 2 == 1),
                 out_dtype=(jnp.float32 if i == 5 else jnp.bfloat16))

    feats = x[..., :3].reshape(_B, _N * 3)
    out = _decoder(feats, W_dec, b_dec.reshape(1, _N * 3))
    return out.reshape(_B, _N, 3)
